# baseline (device time: 38301 ns/iter reference)
import jax
import jax.numpy as jnp
from jax import lax
from jax.experimental import pallas as pl
from jax.experimental.pallas import tpu as pltpu

W = 16
M = 256
LOG_M = 8
HN = 64

_ROUNDS = [
    ("single", 1, 2, True),
    ("pair", 1, 4, True),
    ("pair", 2, 8, False),
    ("single", 1, 8, True),
    ("pair", 4, 16, False),
    ("pair", 1, 16, True),
]
_SLOT_BASE = []
_ns = 0
for _r in _ROUNDS:
    _SLOT_BASE.append(_ns)
    _ns += 3 if _r[0] == "pair" else 1
_N_SLOTS = _ns
_PEERS = sorted({d * q for k, q, _, _ in _ROUNDS
                 for d in ((1, 2, 3) if k == "pair" else (1,))})


def _ce_stage(x, gi_base, j, k):
    r = x.shape[0]
    ii = lax.broadcasted_iota(jnp.int32, x.shape, 0) + gi_base
    lower = (ii & j) == 0
    asc = (ii & k) == 0
    take_min = lower == asc
    up = pltpu.roll(x, r - j, 0)
    dn = pltpu.roll(x, j, 0)
    partner = jnp.where(lower, up, dn)
    return jnp.where(take_min, jnp.minimum(x, partner), jnp.maximum(x, partner))


def _mm(ii, sp, p_blk, a, b):
    tm = ((ii & (sp * M)) == 0) == ((ii & (p_blk * M)) == 0)
    return jnp.where(tm, jnp.minimum(a, b), jnp.maximum(a, b))


def kernel(x):
    m, n = x.shape
    assert m == M and n == 2 * HN

    def body(x_ref, out_ref, cur_refs, recv_refs, send_sems, recv_sems):
        my = lax.axis_index("i")
        gi = my * M

        def local_sort(v):
            for lk in range(1, LOG_M + 1):
                k = 1 << lk
                for lj in range(lk - 1, -1, -1):
                    v = _ce_stage(v, gi, 1 << lj, k)
            return v

        def send_round(s, r):
            kind, q, _, _ = _ROUNDS[r]
            base = _SLOT_BASE[r]
            deltas = (q, 2 * q, 3 * q) if kind == "pair" else (q,)
            rdmas = []
            for idx, d in enumerate(deltas):
                rdma = pltpu.make_async_remote_copy(
                    src_ref=cur_refs.at[s],
                    dst_ref=recv_refs.at[s, base + idx],
                    send_sem=send_sems.at[s, base + idx],
                    recv_sem=recv_sems.at[s, base + idx],
                    device_id=(my ^ d,),
                    device_id_type=pl.DeviceIdType.MESH,
                )
                rdma.start()
                rdmas.append(rdma)
            return rdmas

        def finish_round(s, r, rdmas):
            for rdma in rdmas:
                rdma.wait()
            kind, q, p_blk, phase_end = _ROUNDS[r]
            base = _SLOT_BASE[r]
            ii = lax.broadcasted_iota(jnp.int32, (M, HN), 0) + gi
            if kind == "single":
                res = _mm(ii, q, p_blk, cur_refs[s], recv_refs[s, base])
            else:
                s0 = cur_refs[s]
                s1 = recv_refs[s, base]
                s2 = recv_refs[s, base + 1]
                s3 = recv_refs[s, base + 2]
                a_my = _mm(ii, 2 * q, p_blk, s0, s2)
                a_q = _mm(ii ^ (q * M), 2 * q, p_blk, s1, s3)
                res = _mm(ii, q, p_blk, a_my, a_q)
            if phase_end:
                k = p_blk * M
                for lj in range(LOG_M - 1, -1, -1):
                    res = _ce_stage(res, gi, 1 << lj, k)
            cur_refs[s] = res

        cur_refs[0] = local_sort(x_ref[:, 0:HN].astype(jnp.bfloat16))
        barrier = pltpu.get_barrier_semaphore()
        for d in _PEERS:
            pl.semaphore_signal(
                barrier, inc=1,
                device_id=(my ^ d,),
                device_id_type=pl.DeviceIdType.MESH,
            )
        pl.semaphore_wait(barrier, len(_PEERS))

        in_flight = [None, None]
        in_flight[0] = send_round(0, 0)
        cur_refs[1] = local_sort(x_ref[:, HN:].astype(jnp.bfloat16))
        in_flight[1] = send_round(1, 0)

        n_r = len(_ROUNDS)
        for r in range(n_r):
            finish_round(0, r, in_flight[0])
            if r + 1 < n_r:
                in_flight[0] = send_round(0, r + 1)
            finish_round(1, r, in_flight[1])
            if r + 1 < n_r:
                in_flight[1] = send_round(1, r + 1)

        out_ref[...] = jnp.concatenate(
            [cur_refs[0], cur_refs[1]], axis=1
        ).astype(jnp.float32)

    return pl.pallas_call(
        body,
        out_shape=jax.ShapeDtypeStruct((M, n), jnp.float32),
        in_specs=[pl.BlockSpec(memory_space=pltpu.VMEM)],
        out_specs=pl.BlockSpec(memory_space=pltpu.VMEM),
        scratch_shapes=[
            pltpu.VMEM((2, M, HN), jnp.bfloat16),
            pltpu.VMEM((2, _N_SLOTS, M, HN), jnp.bfloat16),
            pltpu.SemaphoreType.DMA((2, _N_SLOTS)),
            pltpu.SemaphoreType.DMA((2, _N_SLOTS)),
        ],
        compiler_params=pltpu.CompilerParams(collective_id=7),
    )(x)


# device time: 31102 ns/iter; 1.2315x vs baseline; 1.2315x over previous
import jax
import jax.numpy as jnp
from jax import lax
from jax.experimental import pallas as pl
from jax.experimental.pallas import tpu as pltpu

W = 16
M = 256
LOG_M = 8

_ROUNDS = [
    ("single", 1, 2, True),
    ("pair", 1, 4, True),
    ("pair", 2, 8, False),
    ("single", 1, 8, True),
    ("pair", 4, 16, False),
    ("pair", 1, 16, True),
]
_N_SLOTS = sum(3 if r[0] == "pair" else 1 for r in _ROUNDS)
_PEERS = sorted({d * q for k, q, _, _ in _ROUNDS
                 for d in ((1, 2, 3) if k == "pair" else (1,))})


def _ce_stage(x, gi_base, j, k):
    r = x.shape[0]
    ii = lax.broadcasted_iota(jnp.int32, x.shape, 0) + gi_base
    lower = (ii & j) == 0
    asc = (ii & k) == 0
    take_min = lower == asc
    up = pltpu.roll(x, r - j, 0)
    dn = pltpu.roll(x, j, 0)
    partner = jnp.where(lower, up, dn)
    return jnp.where(take_min, jnp.minimum(x, partner), jnp.maximum(x, partner))


def _mm(ii, sp, p_blk, a, b):
    tm = ((ii & (sp * M)) == 0) == ((ii & (p_blk * M)) == 0)
    return jnp.where(tm, jnp.minimum(a, b), jnp.maximum(a, b))


def kernel(x):
    m, n = x.shape
    assert m == M

    def body(x_ref, out_ref, cur_ref, recv_ref, send_sems, recv_sems):
        my = lax.axis_index("i")
        gi = my * M

        v = x_ref[...].astype(jnp.bfloat16)
        for lk in range(1, LOG_M + 1):
            k = 1 << lk
            for lj in range(lk - 1, -1, -1):
                v = _ce_stage(v, gi, 1 << lj, k)
        cur_ref[...] = v

        barrier = pltpu.get_barrier_semaphore()
        for d in _PEERS:
            pl.semaphore_signal(
                barrier, inc=1,
                device_id=(my ^ d,),
                device_id_type=pl.DeviceIdType.MESH,
            )
        pl.semaphore_wait(barrier, len(_PEERS))

        base = 0
        for kind, q, p_blk, phase_end in _ROUNDS:
            k = p_blk * M
            deltas = (q, 2 * q, 3 * q) if kind == "pair" else (q,)
            rdmas = []
            for idx, d in enumerate(deltas):
                rdma = pltpu.make_async_remote_copy(
                    src_ref=cur_ref,
                    dst_ref=recv_ref.at[base + idx],
                    send_sem=send_sems.at[base + idx],
                    recv_sem=recv_sems.at[base + idx],
                    device_id=(my ^ d,),
                    device_id_type=pl.DeviceIdType.MESH,
                )
                rdma.start()
                rdmas.append(rdma)
            for rdma in rdmas:
                rdma.wait()

            ii = lax.broadcasted_iota(jnp.int32, (M, n), 0) + gi
            if kind == "single":
                cur_ref[...] = _mm(ii, q, p_blk, cur_ref[...], recv_ref[base])
            else:
                s0 = cur_ref[...]
                s1 = recv_ref[base]
                s2 = recv_ref[base + 1]
                s3 = recv_ref[base + 2]
                a_my = _mm(ii, 2 * q, p_blk, s0, s2)
                a_q = _mm(ii ^ (q * M), 2 * q, p_blk, s1, s3)
                cur_ref[...] = _mm(ii, q, p_blk, a_my, a_q)
            base += len(deltas)

            if phase_end:
                v = cur_ref[...]
                for lj in range(LOG_M - 1, -1, -1):
                    v = _ce_stage(v, gi, 1 << lj, k)
                cur_ref[...] = v

        out_ref[...] = cur_ref[...].astype(jnp.float32)

    return pl.pallas_call(
        body,
        out_shape=jax.ShapeDtypeStruct((M, n), jnp.float32),
        in_specs=[pl.BlockSpec(memory_space=pltpu.VMEM)],
        out_specs=pl.BlockSpec(memory_space=pltpu.VMEM),
        scratch_shapes=[
            pltpu.VMEM((M, n), jnp.bfloat16),
            pltpu.VMEM((_N_SLOTS, M, n), jnp.bfloat16),
            pltpu.SemaphoreType.DMA((_N_SLOTS,)),
            pltpu.SemaphoreType.DMA((_N_SLOTS,)),
        ],
        compiler_params=pltpu.CompilerParams(collective_id=7),
    )(x)


# device time: 30999 ns/iter; 1.2356x vs baseline; 1.0033x over previous
import jax
import jax.numpy as jnp
from jax import lax
from jax.experimental import pallas as pl
from jax.experimental.pallas import tpu as pltpu

W = 16
M = 256
LOG_M = 8

_ROUNDS = [
    ("single", 1, 2, True),
    ("pair", 1, 4, True),
    ("pair", 2, 8, False),
    ("single", 1, 8, True),
    ("pair", 4, 16, False),
    ("pair", 1, 16, True),
]
_N_SLOTS = sum(3 if r[0] == "pair" else 1 for r in _ROUNDS)
_PEERS = sorted({d * q for k, q, _, _ in _ROUNDS
                 for d in ((1, 2, 3) if k == "pair" else (1,))})


def _ce_stage(x, gi_base, j, k):
    r, n = x.shape
    if j >= 16:
        m = r // (2 * j)
        x4 = x.reshape(m, 2, j, n)
        a = x4[:, 0]
        b = x4[:, 1]
        mn = jnp.minimum(a, b)
        mx = jnp.maximum(a, b)
        ii = lax.broadcasted_iota(jnp.int32, (m, j, n), 0) * (2 * j) + gi_base
        asc = (ii & k) == 0
        na = jnp.where(asc, mn, mx)
        nb = jnp.where(asc, mx, mn)
        return jnp.stack([na, nb], axis=1).reshape(r, n)
    ii = lax.broadcasted_iota(jnp.int32, x.shape, 0) + gi_base
    lower = (ii & j) == 0
    asc = (ii & k) == 0
    take_min = lower == asc
    up = pltpu.roll(x, r - j, 0)
    dn = pltpu.roll(x, j, 0)
    partner = jnp.where(lower, up, dn)
    return jnp.where(take_min, jnp.minimum(x, partner), jnp.maximum(x, partner))


def _mm(ii, sp, p_blk, a, b):
    tm = ((ii & (sp * M)) == 0) == ((ii & (p_blk * M)) == 0)
    return jnp.where(tm, jnp.minimum(a, b), jnp.maximum(a, b))


def kernel(x):
    m, n = x.shape
    assert m == M

    def body(x_ref, out_ref, cur_ref, recv_ref, send_sems, recv_sems):
        my = lax.axis_index("i")
        gi = my * M

        v = x_ref[...].astype(jnp.bfloat16)
        for lk in range(1, LOG_M + 1):
            k = 1 << lk
            for lj in range(lk - 1, -1, -1):
                v = _ce_stage(v, gi, 1 << lj, k)
        cur_ref[...] = v

        barrier = pltpu.get_barrier_semaphore()
        for d in _PEERS:
            pl.semaphore_signal(
                barrier, inc=1,
                device_id=(my ^ d,),
                device_id_type=pl.DeviceIdType.MESH,
            )
        pl.semaphore_wait(barrier, len(_PEERS))

        base = 0
        for kind, q, p_blk, phase_end in _ROUNDS:
            k = p_blk * M
            deltas = (q, 2 * q, 3 * q) if kind == "pair" else (q,)
            rdmas = []
            for idx, d in enumerate(deltas):
                rdma = pltpu.make_async_remote_copy(
                    src_ref=cur_ref,
                    dst_ref=recv_ref.at[base + idx],
                    send_sem=send_sems.at[base + idx],
                    recv_sem=recv_sems.at[base + idx],
                    device_id=(my ^ d,),
                    device_id_type=pl.DeviceIdType.MESH,
                )
                rdma.start()
                rdmas.append(rdma)
            for rdma in rdmas:
                rdma.wait()

            ii = lax.broadcasted_iota(jnp.int32, (M, n), 0) + gi
            if kind == "single":
                cur_ref[...] = _mm(ii, q, p_blk, cur_ref[...], recv_ref[base])
            else:
                s0 = cur_ref[...]
                s1 = recv_ref[base]
                s2 = recv_ref[base + 1]
                s3 = recv_ref[base + 2]
                a_my = _mm(ii, 2 * q, p_blk, s0, s2)
                a_q = _mm(ii ^ (q * M), 2 * q, p_blk, s1, s3)
                cur_ref[...] = _mm(ii, q, p_blk, a_my, a_q)
            base += len(deltas)

            if phase_end:
                v = cur_ref[...]
                for lj in range(LOG_M - 1, -1, -1):
                    v = _ce_stage(v, gi, 1 << lj, k)
                cur_ref[...] = v

        out_ref[...] = cur_ref[...].astype(jnp.float32)

    return pl.pallas_call(
        body,
        out_shape=jax.ShapeDtypeStruct((M, n), jnp.float32),
        in_specs=[pl.BlockSpec(memory_space=pltpu.VMEM)],
        out_specs=pl.BlockSpec(memory_space=pltpu.VMEM),
        scratch_shapes=[
            pltpu.VMEM((M, n), jnp.bfloat16),
            pltpu.VMEM((_N_SLOTS, M, n), jnp.bfloat16),
            pltpu.SemaphoreType.DMA((_N_SLOTS,)),
            pltpu.SemaphoreType.DMA((_N_SLOTS,)),
        ],
        compiler_params=pltpu.CompilerParams(collective_id=7),
    )(x)
